# baseline (speedup 1.0000x reference)
import functools

import jax
import jax.numpy as jnp
from jax import lax
from jax.experimental import pallas as pl
from jax.experimental.pallas import tpu as pltpu
from jax.experimental.pallas import tpu_sc as plsc

N = 10000
E = 320000
D = 128
C = 4
BN_EPS = 1e-5

NC = 2
NS = 16
NW = NC * NS
EPT = E // NW
K = 40
NCHUNK = EPT // K
NB = 3
RPT = 624
REM = N - NS * RPT
LANES = 16
G = D // LANES
G2 = D // 32
DH = D // 2


def _sc_edge_body(x_hbm, src_hbm, dst_hbm, ea_hbm, out_hbm, aggr_sh, src_t,
                  xb0, xb1, xb2, eb0, eb1, eb2, sb0, sb1, sb2, dv0, dv1, dv2,
                  gs0, gs1, gs2, es0, es1, es2, is0, is1, is2,
                  ss0, ss1, ss2):
    xb = (xb0, xb1, xb2)
    eb = (eb0, eb1, eb2)
    sb = (sb0, sb1, sb2)
    dv = (dv0, dv1, dv2)
    gs = (gs0, gs1, gs2)
    es = (es0, es1, es2)
    isem = (is0, is1, is2)
    ss = (ss0, ss1, ss2)
    c = lax.axis_index("c")
    s = lax.axis_index("s")
    wid = c * NS + s
    tile_base = wid * EPT

    def zrow(r, carry):
        for g in range(G):
            sb0[r, pl.ds(LANES * g, LANES)] = jnp.zeros((LANES,), jnp.float32)
        return carry
    lax.fori_loop(0, K, zrow, 0)
    for j in range(RPT // K):
        pltpu.sync_copy(sb0, aggr_sh.at[pl.ds(s * RPT + j * K, K)])
    ZREM = RPT - (RPT // K) * K
    pltpu.sync_copy(sb0.at[pl.ds(0, ZREM)],
                    aggr_sh.at[pl.ds(s * RPT + (RPT // K) * K, ZREM)])

    @pl.when(s == NS - 1)
    def _zero_rem():
        pltpu.sync_copy(sb0.at[pl.ds(0, REM)], aggr_sh.at[pl.ds(NS * RPT, REM)])

    pltpu.sync_copy(src_hbm.at[pl.ds(tile_base, EPT)], src_t)
    plsc.subcore_barrier()

    def issue(i, j, guard):
        @pl.when(i < NCHUNK)
        def _():
            if guard:
                pltpu.make_async_copy(sb[j], aggr_sh.at[dv[j]], ss[j]).wait()
            base = tile_base + i * K
            pltpu.async_copy(dst_hbm.at[pl.ds(base, K)], dv[j], isem[j])
            pltpu.async_copy(ea_hbm.at[pl.ds(base * DH, K * DH)], eb[j], es[j])
            pltpu.async_copy(x_hbm.at[src_t.at[pl.ds(i * K, K)]], xb[j], gs[j])

    def consume(i, j):
        base = tile_base + i * K
        pltpu.make_async_copy(x_hbm.at[src_t.at[pl.ds(i * K, K)]],
                              xb[j], gs[j]).wait()
        pltpu.make_async_copy(ea_hbm.at[pl.ds(base * DH, K * DH)],
                              eb[j], es[j]).wait()
        pltpu.make_async_copy(dst_hbm.at[pl.ds(base, K)], dv[j], isem[j]).wait()

        def row(r, rcarry):
            for g in range(G2):
                ew = eb[j][pl.ds(r * DH + LANES * g, LANES)]
                elo = lax.bitcast_convert_type(
                    jnp.left_shift(ew, 16), jnp.float32)
                ehi = lax.bitcast_convert_type(
                    jnp.bitwise_and(ew, jnp.int32(-65536)), jnp.float32)
                a = pl.ds(LANES * g, LANES)
                b = pl.ds(DH + LANES * g, LANES)
                sb[j][r, a] = jnp.maximum(xb[j][r, a] + elo, 0.0)
                sb[j][r, b] = jnp.maximum(xb[j][r, b] + ehi, 0.0)
            return rcarry
        lax.fori_loop(0, K, row, 0)
        pltpu.async_copy(sb[j], aggr_sh.at[dv[j]], ss[j], add=True)

    issue(0, 0, False)
    issue(1, 1, False)
    issue(2, 2, False)
    consume(0, 0)
    issue(3, 0, True)
    consume(1, 1)
    issue(4, 1, True)
    consume(2, 2)
    issue(5, 2, True)

    def block(t, carry):
        i = 3 * t
        for k in range(3):
            consume(i + k, k)
            issue(i + k + 3, k, True)
        return carry
    lax.fori_loop(1, NCHUNK // 3, block, 0)
    consume(NCHUNK - 1, 0)
    for j in range(NB):
        pltpu.make_async_copy(sb[j], aggr_sh.at[dv[j]], ss[j]).wait()

    plsc.subcore_barrier()
    pltpu.sync_copy(aggr_sh.at[pl.ds(s * RPT, RPT)],
                    out_hbm.at[c, pl.ds(s * RPT, RPT)])

    @pl.when(s == NS - 1)
    def _copy_rem():
        pltpu.sync_copy(aggr_sh.at[pl.ds(NS * RPT, REM)],
                        out_hbm.at[c, pl.ds(NS * RPT, REM)])


_sc_edge = functools.partial(
    pl.kernel,
    mesh=plsc.VectorSubcoreMesh(core_axis_name="c", subcore_axis_name="s"),
    out_type=jax.ShapeDtypeStruct((NC, N, D), jnp.float32),
    scratch_types=(
        [pltpu.VMEM_SHARED((N, D), jnp.float32)]
        + [pltpu.VMEM((EPT,), jnp.int32)]
        + [pltpu.VMEM((K, D), jnp.float32)] * 3
        + [pltpu.VMEM((K * DH,), jnp.int32)] * 3
        + [pltpu.VMEM((K, D), jnp.float32)] * 3
        + [pltpu.VMEM((K,), jnp.int32)] * 3
        + [pltpu.SemaphoreType.DMA] * 12
    ),
)(_sc_edge_body)


def _tc_body(final, x_ref, aggr_ref, w1_ref, b1_ref, w2_ref, b2_ref,
             mask_ref, gamma_ref, beta_ref, xin_ref, eps_ref, out_ref):
    x = x_ref[...]
    a = aggr_ref[0] + aggr_ref[1]
    h = (1.0 + eps_ref[0, 0]) * x + a
    h = jnp.maximum(jnp.dot(h, w1_ref[...],
                            preferred_element_type=jnp.float32) + b1_ref[...], 0.0)
    h = jnp.dot(h, w2_ref[...], preferred_element_type=jnp.float32) + b2_ref[...]
    y = mask_ref[...] * h + x
    mu = jnp.mean(y, axis=0, keepdims=True)
    var = jnp.mean((y - mu) * (y - mu), axis=0, keepdims=True)
    y = gamma_ref[...] * (y - mu) * lax.rsqrt(var + BN_EPS) + beta_ref[...]
    if final:
        y = xin_ref[...] + jnp.maximum(y, 0.0)
    out_ref[...] = y


def _tc_update(x, aggr2, w1, b1, w2, b2, mask, gamma, beta, x_in, eps_c, final):
    return pl.pallas_call(
        functools.partial(_tc_body, final),
        out_shape=jax.ShapeDtypeStruct((N, D), jnp.float32),
        in_specs=[pl.BlockSpec(memory_space=pltpu.VMEM)] * 10
        + [pl.BlockSpec(memory_space=pltpu.SMEM)],
    )(x, aggr2, w1, b1, w2, b2, mask, gamma, beta, x_in, eps_c)


import numpy as np

_PERM = np.concatenate([np.arange(0, D, 2), np.arange(1, D, 2)])
_IPERM = np.argsort(_PERM)


def _pack16(a):
    m = a.shape[0]
    a16 = a.astype(jnp.bfloat16).reshape(m, DH, 2)
    return jax.lax.bitcast_convert_type(a16, jnp.int32).reshape(m * DH)


def kernel(x, edge_index, edge_attr, masks, complement_masks,
           W1, b1, W2, b2, eps, gamma, beta):
    src = edge_index[0]
    dst = edge_index[1]
    ea16 = _pack16(edge_attr)
    x = x[:, _PERM]
    x_in = x
    for c in range(C):
        aggr2 = _sc_edge(x, src, dst, ea16)
        x = _tc_update(
            x, aggr2, W1[c][_PERM][:, _PERM], b1[c][_PERM].reshape(1, D),
            W2[c][_PERM][:, _PERM], b2[c][_PERM].reshape(1, D),
            masks[c].reshape(N, 1), gamma[c][_PERM].reshape(1, D),
            beta[c][_PERM].reshape(1, D),
            x_in, eps[c].reshape(1, 1), final=(c == C - 1))
    return x[:, _IPERM]


# speedup vs baseline: 3.0671x; 3.0671x over previous
import functools

import jax
import jax.numpy as jnp
from jax import lax
from jax.experimental import pallas as pl
from jax.experimental.pallas import tpu as pltpu
from jax.experimental.pallas import tpu_sc as plsc

N = 10000
E = 320000
D = 128
C = 4
BN_EPS = 1e-5

NC = 2
NS = 16
NW = NC * NS
EPT = E // NW
K = 40
NCHUNK = EPT // K
NB = 3
RPT = 624
REM = N - NS * RPT
LANES = 16
G = D // LANES


def _sc_edge_body(x_hbm, src_hbm, dst_hbm, ea_hbm, out_hbm, aggr_sh, src_t,
                  xb0, xb1, xb2, eb0, eb1, eb2, dv0, dv1, dv2,
                  gs0, gs1, gs2, es0, es1, es2, is0, is1, is2,
                  ss0, ss1, ss2):
    xb = (xb0, xb1, xb2)
    eb = (eb0, eb1, eb2)
    dv = (dv0, dv1, dv2)
    gs = (gs0, gs1, gs2)
    es = (es0, es1, es2)
    isem = (is0, is1, is2)
    ss = (ss0, ss1, ss2)
    c = lax.axis_index("c")
    s = lax.axis_index("s")
    wid = c * NS + s
    tile_base = wid * EPT

    def zrow(r, carry):
        for g in range(G):
            eb0[r, pl.ds(LANES * g, LANES)] = jnp.zeros((LANES,), jnp.float32)
        return carry
    lax.fori_loop(0, K, zrow, 0)
    for j in range(RPT // K):
        pltpu.sync_copy(eb0, aggr_sh.at[pl.ds(s * RPT + j * K, K)])
    ZREM = RPT - (RPT // K) * K
    pltpu.sync_copy(eb0.at[pl.ds(0, ZREM)],
                    aggr_sh.at[pl.ds(s * RPT + (RPT // K) * K, ZREM)])

    @pl.when(s == NS - 1)
    def _zero_rem():
        pltpu.sync_copy(eb0.at[pl.ds(0, REM)], aggr_sh.at[pl.ds(NS * RPT, REM)])

    pltpu.sync_copy(src_hbm.at[pl.ds(tile_base, EPT)], src_t)
    plsc.subcore_barrier()

    def issue(i, j, guard):
        @pl.when(i < NCHUNK)
        def _():
            if guard:
                pltpu.make_async_copy(eb[j], aggr_sh.at[dv[j]], ss[j]).wait()
            base = tile_base + i * K
            pltpu.async_copy(dst_hbm.at[pl.ds(base, K)], dv[j], isem[j])
            pltpu.async_copy(ea_hbm.at[pl.ds(base, K)], eb[j], es[j])
            pltpu.async_copy(x_hbm.at[src_t.at[pl.ds(i * K, K)]], xb[j], gs[j])

    def consume(i, j):
        base = tile_base + i * K
        pltpu.make_async_copy(x_hbm.at[src_t.at[pl.ds(i * K, K)]],
                              xb[j], gs[j]).wait()
        pltpu.make_async_copy(ea_hbm.at[pl.ds(base, K)], eb[j], es[j]).wait()
        pltpu.make_async_copy(dst_hbm.at[pl.ds(base, K)], dv[j], isem[j]).wait()

        def row(r, rcarry):
            for g in range(G):
                sl = pl.ds(LANES * g, LANES)
                eb[j][r, sl] = jnp.maximum(xb[j][r, sl] + eb[j][r, sl], 0.0)
            return rcarry
        lax.fori_loop(0, K, row, 0)
        pltpu.async_copy(eb[j], aggr_sh.at[dv[j]], ss[j], add=True)

    issue(0, 0, False)
    issue(1, 1, False)
    issue(2, 2, False)
    consume(0, 0)
    issue(3, 0, True)
    consume(1, 1)
    issue(4, 1, True)
    consume(2, 2)
    issue(5, 2, True)

    def block(t, carry):
        i = 3 * t
        for k in range(3):
            consume(i + k, k)
            issue(i + k + 3, k, True)
        return carry
    lax.fori_loop(1, NCHUNK // 3, block, 0)
    consume(NCHUNK - 1, 0)
    for j in range(NB):
        pltpu.make_async_copy(eb[j], aggr_sh.at[dv[j]], ss[j]).wait()

    plsc.subcore_barrier()
    pltpu.sync_copy(aggr_sh.at[pl.ds(s * RPT, RPT)],
                    out_hbm.at[c, pl.ds(s * RPT, RPT)])

    @pl.when(s == NS - 1)
    def _copy_rem():
        pltpu.sync_copy(aggr_sh.at[pl.ds(NS * RPT, REM)],
                        out_hbm.at[c, pl.ds(NS * RPT, REM)])


_sc_edge = functools.partial(
    pl.kernel,
    mesh=plsc.VectorSubcoreMesh(core_axis_name="c", subcore_axis_name="s"),
    out_type=jax.ShapeDtypeStruct((NC, N, D), jnp.float32),
    scratch_types=(
        [pltpu.VMEM_SHARED((N, D), jnp.float32)]
        + [pltpu.VMEM((EPT,), jnp.int32)]
        + [pltpu.VMEM((K, D), jnp.float32)] * 3
        + [pltpu.VMEM((K, D), jnp.float32)] * 3
        + [pltpu.VMEM((K,), jnp.int32)] * 3
        + [pltpu.SemaphoreType.DMA] * 12
    ),
)(_sc_edge_body)


def _tc_body(final, x_ref, aggr_ref, w1_ref, b1_ref, w2_ref, b2_ref,
             mask_ref, gamma_ref, beta_ref, xin_ref, eps_ref, out_ref):
    x = x_ref[...]
    a = aggr_ref[0] + aggr_ref[1]
    h = (1.0 + eps_ref[0, 0]) * x + a
    h = jnp.maximum(jnp.dot(h, w1_ref[...],
                            preferred_element_type=jnp.float32) + b1_ref[...], 0.0)
    h = jnp.dot(h, w2_ref[...], preferred_element_type=jnp.float32) + b2_ref[...]
    y = mask_ref[...] * h + x
    mu = jnp.mean(y, axis=0, keepdims=True)
    var = jnp.mean((y - mu) * (y - mu), axis=0, keepdims=True)
    y = gamma_ref[...] * (y - mu) * lax.rsqrt(var + BN_EPS) + beta_ref[...]
    if final:
        y = xin_ref[...] + jnp.maximum(y, 0.0)
    out_ref[...] = y


def _tc_update(x, aggr2, w1, b1, w2, b2, mask, gamma, beta, x_in, eps_c, final):
    return pl.pallas_call(
        functools.partial(_tc_body, final),
        out_shape=jax.ShapeDtypeStruct((N, D), jnp.float32),
        in_specs=[pl.BlockSpec(memory_space=pltpu.VMEM)] * 10
        + [pl.BlockSpec(memory_space=pltpu.SMEM)],
    )(x, aggr2, w1, b1, w2, b2, mask, gamma, beta, x_in, eps_c)


def kernel(x, edge_index, edge_attr, masks, complement_masks,
           W1, b1, W2, b2, eps, gamma, beta):
    src = edge_index[0]
    dst = edge_index[1]
    x_in = x
    for c in range(C):
        aggr2 = _sc_edge(x, src, dst, edge_attr)
        x = _tc_update(
            x, aggr2, W1[c], b1[c].reshape(1, D), W2[c], b2[c].reshape(1, D),
            masks[c].reshape(N, 1), gamma[c].reshape(1, D), beta[c].reshape(1, D),
            x_in, eps[c].reshape(1, 1), final=(c == C - 1))
    return x
